# baseline (speedup 1.0000x reference)
import functools

import jax
import jax.numpy as jnp
from jax import lax
from jax.experimental import pallas as pl
from jax.experimental.pallas import tpu as pltpu
from jax.experimental.pallas import tpu_sc as plsc

D_MODEL = 768
D_FF = 3072
N_EXPERTS = 8
SEQ = 2048
LANES = 128
CHUNKS = SEQ // LANES
TILE = 256
N_ASSIGN = SEQ * 2
PADDED = N_ASSIGN + N_EXPERTS * TILE
NT = PADDED // TILE
NEG = -1e30

NW = 32
ROWS_PER_W = SEQ // NW
VREGS_PER_ROW = D_MODEL // 16


def _router_body(x_ref, gw_ref, gb_ref,
                 pos0_ref, pos1_ref, w0_ref, w1_ref, counts_ref, loss_ref):
    x = x_ref[...]
    logits2 = jnp.dot(x, gw_ref[...], preferred_element_type=jnp.float32)
    logits2 = logits2 + gb_ref[...]
    lg = logits2.reshape(CHUNKS, LANES, LANES)
    lane = lax.broadcasted_iota(jnp.int32, (CHUNKS, LANES, LANES), 2)
    valid = lane < N_EXPERTS
    lg = jnp.where(valid, lg, NEG)

    m0 = jnp.max(lg, axis=2)
    is0 = (lg == m0[:, :, None]) & valid
    i0 = jnp.min(jnp.where(is0, lane, N_EXPERTS + 1), axis=2)
    oh0 = lane == i0[:, :, None]
    lg1 = jnp.where(oh0, NEG, lg)
    m1 = jnp.max(lg1, axis=2)
    is1 = (lg1 == m1[:, :, None]) & valid
    i1 = jnp.min(jnp.where(is1, lane, N_EXPERTS + 1), axis=2)
    oh1 = lane == i1[:, :, None]

    e1 = jnp.exp(m1 - m0)
    w0 = 1.0 / (1.0 + e1)
    w0_ref[...] = w0
    w1_ref[...] = 1.0 - w0

    p = jnp.exp(lg - m0[:, :, None])
    p = p / jnp.sum(p, axis=2, keepdims=True)
    usage = jnp.sum(p, axis=(0, 1)) * (1.0 / SEQ)
    loss = N_EXPERTS * jnp.sum(usage * usage) - 1.0
    loss_ref[...] = jnp.full((8, LANES), loss, jnp.float32)

    A0 = oh0.astype(jnp.float32)
    A1 = oh1.astype(jnp.float32)
    C = A0 + A1
    r_idx = lax.broadcasted_iota(jnp.int32, (CHUNKS, LANES, LANES), 1)
    p_idx = lax.broadcasted_iota(jnp.int32, (CHUNKS, LANES, LANES), 2)
    Tl = (p_idx < r_idx).astype(jnp.float32)
    X = lax.dot_general(Tl, C, (((2,), (1,)), ((0,), (0,))),
                        preferred_element_type=jnp.float32)
    tot = jnp.sum(C, axis=1)
    a16 = lax.broadcasted_iota(jnp.int32, (CHUNKS, CHUNKS), 0)
    b16 = lax.broadcasted_iota(jnp.int32, (CHUNKS, CHUNKS), 1)
    T16 = (b16 < a16).astype(jnp.float32)
    Y = lax.dot_general(T16, tot, (((1,), (0,)), ((), ())),
                        preferred_element_type=jnp.float32)
    S = X + Y[:, None, :]

    counts_row = jnp.sum(tot, axis=0)[None, :]
    counts_ref[...] = jnp.broadcast_to(counts_row, (8, LANES))
    pc_row = jnp.ceil(counts_row * (1.0 / TILE)) * TILE
    ag = lax.broadcasted_iota(jnp.int32, (LANES, LANES), 0)
    bg = lax.broadcasted_iota(jnp.int32, (LANES, LANES), 1)
    Tg = (ag < bg).astype(jnp.float32)
    starts_row = lax.dot_general(pc_row, Tg, (((1,), (0,)), ((), ())),
                                 preferred_element_type=jnp.float32)
    base = S + starts_row[0][None, None, :]
    pos0_ref[...] = jnp.sum(base * A0, axis=2).astype(jnp.int32)
    pos1_ref[...] = jnp.sum(base * A1, axis=2).astype(jnp.int32)


def _router(x2, gate_w, gate_b):
    gw = jnp.pad(gate_w, ((0, 0), (0, LANES - N_EXPERTS)))
    gb = jnp.pad(gate_b, (0, LANES - N_EXPERTS))[None, :]
    outs = pl.pallas_call(
        _router_body,
        out_shape=(
            jax.ShapeDtypeStruct((CHUNKS, LANES), jnp.int32),
            jax.ShapeDtypeStruct((CHUNKS, LANES), jnp.int32),
            jax.ShapeDtypeStruct((CHUNKS, LANES), jnp.float32),
            jax.ShapeDtypeStruct((CHUNKS, LANES), jnp.float32),
            jax.ShapeDtypeStruct((8, LANES), jnp.float32),
            jax.ShapeDtypeStruct((8, LANES), jnp.float32),
        ),
    )(x2, gw, gb)
    return outs


def _dispatch_rows(x2, pos0, pos1):
    mesh = plsc.VectorSubcoreMesh(core_axis_name="c", subcore_axis_name="s")

    @functools.partial(
        pl.kernel, mesh=mesh,
        out_type=jax.ShapeDtypeStruct((PADDED, D_MODEL), jnp.float32),
        scratch_types=[
            pltpu.VMEM((ROWS_PER_W,), jnp.int32),
            pltpu.VMEM((ROWS_PER_W,), jnp.int32),
            pltpu.VMEM((ROWS_PER_W, D_MODEL), jnp.float32),
            pltpu.SemaphoreType.DMA,
        ],
    )
    def disp(x_hbm, pos0_hbm, pos1_hbm, xs_hbm, idx0_v, idx1_v, rows_v, sem):
        wid = lax.axis_index("s") * 2 + lax.axis_index("c")
        base = wid * ROWS_PER_W
        pltpu.sync_copy(pos0_hbm.at[pl.ds(base, ROWS_PER_W)], idx0_v)
        pltpu.sync_copy(pos1_hbm.at[pl.ds(base, ROWS_PER_W)], idx1_v)
        pltpu.sync_copy(x_hbm.at[pl.ds(base, ROWS_PER_W), :], rows_v)
        pltpu.async_copy(rows_v, xs_hbm.at[idx0_v], sem).wait()
        pltpu.async_copy(rows_v, xs_hbm.at[idx1_v], sem).wait()

    return disp(x2, pos0, pos1)


def _ffn_body(gid_ref, used_ref, xs_ref, W1_ref, B1_ref, W2_ref, B2_ref,
              ys_ref):
    i = pl.program_id(0)

    @pl.when(i < used_ref[0])
    def _():
        h = jnp.dot(xs_ref[...].astype(jnp.bfloat16), W1_ref[0],
                    preferred_element_type=jnp.float32)
        h = h + B1_ref[0]
        h = 0.5 * h * (1.0 + lax.erf(h * 0.7071067811865476))
        y = jnp.dot(h, W2_ref[0], preferred_element_type=jnp.float32)
        ys_ref[...] = y + B2_ref[0]


def _ffn(xs, W1, B1, W2, B2, gid, used):
    grid_spec = pltpu.PrefetchScalarGridSpec(
        num_scalar_prefetch=2,
        grid=(NT,),
        in_specs=[
            pl.BlockSpec((TILE, D_MODEL), lambda i, g, u: (i, 0)),
            pl.BlockSpec((1, D_MODEL, D_FF), lambda i, g, u: (g[i], 0, 0)),
            pl.BlockSpec((1, 1, D_FF), lambda i, g, u: (g[i], 0, 0)),
            pl.BlockSpec((1, D_FF, D_MODEL), lambda i, g, u: (g[i], 0, 0)),
            pl.BlockSpec((1, 1, D_MODEL), lambda i, g, u: (g[i], 0, 0)),
        ],
        out_specs=pl.BlockSpec((TILE, D_MODEL), lambda i, g, u: (i, 0)),
    )
    return pl.pallas_call(
        _ffn_body,
        grid_spec=grid_spec,
        out_shape=jax.ShapeDtypeStruct((PADDED, D_MODEL), jnp.float32),
        compiler_params=pltpu.CompilerParams(
            dimension_semantics=("arbitrary",)),
    )(gid, used, xs, W1.astype(jnp.bfloat16), B1[:, None, :], W2,
      B2[:, None, :])


def _combine_rows(ys, pos0, pos1, w0, w1):
    w0x = jnp.broadcast_to(w0[:, None], (SEQ, 16))
    w1x = jnp.broadcast_to(w1[:, None], (SEQ, 16))
    mesh = plsc.VectorSubcoreMesh(core_axis_name="c", subcore_axis_name="s")

    @functools.partial(
        pl.kernel, mesh=mesh,
        out_type=jax.ShapeDtypeStruct((SEQ, D_MODEL), jnp.float32),
        scratch_types=[
            pltpu.VMEM((ROWS_PER_W,), jnp.int32),
            pltpu.VMEM((ROWS_PER_W,), jnp.int32),
            pltpu.VMEM((ROWS_PER_W, 16), jnp.float32),
            pltpu.VMEM((ROWS_PER_W, 16), jnp.float32),
            pltpu.VMEM((ROWS_PER_W, D_MODEL), jnp.float32),
            pltpu.VMEM((ROWS_PER_W, D_MODEL), jnp.float32),
            pltpu.SemaphoreType.DMA,
        ],
    )
    def comb(ys_hbm, pos0_hbm, pos1_hbm, w0_hbm, w1_hbm, out_hbm,
             i0_v, i1_v, w0_v, w1_v, r0_v, r1_v, sem):
        wid = lax.axis_index("s") * 2 + lax.axis_index("c")
        base = wid * ROWS_PER_W
        pltpu.sync_copy(pos0_hbm.at[pl.ds(base, ROWS_PER_W)], i0_v)
        pltpu.sync_copy(pos1_hbm.at[pl.ds(base, ROWS_PER_W)], i1_v)
        pltpu.sync_copy(w0_hbm.at[pl.ds(base, ROWS_PER_W), :], w0_v)
        pltpu.sync_copy(w1_hbm.at[pl.ds(base, ROWS_PER_W), :], w1_v)
        pltpu.async_copy(ys_hbm.at[i0_v], r0_v, sem).wait()
        pltpu.async_copy(ys_hbm.at[i1_v], r1_v, sem).wait()

        def body(m, carry):
            a = w0_v[m, :]
            b = w1_v[m, :]
            for j in range(VREGS_PER_ROW):
                s = 16 * j
                r0_v[m, pl.ds(s, 16)] = (a * r0_v[m, pl.ds(s, 16)]
                                         + b * r1_v[m, pl.ds(s, 16)])
            return carry

        lax.fori_loop(0, ROWS_PER_W, body, 0)
        pltpu.sync_copy(r0_v, out_hbm.at[pl.ds(base, ROWS_PER_W), :])

    return comb(ys, pos0, pos1, w0x, w1x)


def kernel(x, gate_w, gate_b, W1, B1, W2, B2):
    x2 = x.reshape(SEQ, D_MODEL)
    pos0, pos1, w0, w1, counts_m, loss_m = _router(x2, gate_w, gate_b)
    pos0 = pos0.reshape(N_ASSIGN // 2)
    pos1 = pos1.reshape(N_ASSIGN // 2)
    w0 = w0.reshape(N_ASSIGN // 2)
    w1 = w1.reshape(N_ASSIGN // 2)

    counts = counts_m[0, :N_EXPERTS].astype(jnp.int32)
    pc = ((counts + TILE - 1) // TILE) * TILE
    ps = jnp.cumsum(pc)
    offs = jnp.arange(NT, dtype=jnp.int32) * TILE
    gid = jnp.sum((ps[None, :] <= offs[:, None]).astype(jnp.int32), axis=1)
    last = jnp.max(jnp.where(pc > 0, jnp.arange(N_EXPERTS), -1))
    gid = jnp.where(offs < ps[-1], gid, last).astype(jnp.int32)
    used = (ps[-1] // TILE).astype(jnp.int32).reshape(1)

    xs = _dispatch_rows(x2, pos0, pos1)
    ys = _ffn(xs, W1, B1, W2, B2, gid, used)
    out = _combine_rows(ys, pos0, pos1, w0, w1)
    return out.reshape(1, SEQ, D_MODEL), loss_m[0, 0]


# speedup vs baseline: 1.2078x; 1.2078x over previous
import functools

import jax
import jax.numpy as jnp
from jax import lax
from jax.experimental import pallas as pl
from jax.experimental.pallas import tpu as pltpu
from jax.experimental.pallas import tpu_sc as plsc

D_MODEL = 768
D_FF = 3072
N_EXPERTS = 8
SEQ = 2048
LANES = 128
CHUNKS = SEQ // LANES
TILE = 256
N_ASSIGN = SEQ * 2
PADDED = N_ASSIGN + N_EXPERTS * TILE
NT = PADDED // TILE
NEG = -1e30

NW = 32
ROWS_PER_W = SEQ // NW
VREGS_PER_ROW = D_MODEL // 16


def _router_body(x_ref, gw_ref, gb_ref,
                 pos0_ref, pos1_ref, w0_ref, w1_ref, meta_ref, loss_ref):
    x = x_ref[...]
    logits2 = jnp.dot(x, gw_ref[...], preferred_element_type=jnp.float32)
    logits2 = logits2 + gb_ref[...]
    lg = logits2.reshape(CHUNKS, LANES, LANES)
    lane = lax.broadcasted_iota(jnp.int32, (CHUNKS, LANES, LANES), 2)
    valid = lane < N_EXPERTS
    lg = jnp.where(valid, lg, NEG)

    m0 = jnp.max(lg, axis=2)
    is0 = (lg == m0[:, :, None]) & valid
    i0 = jnp.min(jnp.where(is0, lane, N_EXPERTS + 1), axis=2)
    oh0 = lane == i0[:, :, None]
    lg1 = jnp.where(oh0, NEG, lg)
    m1 = jnp.max(lg1, axis=2)
    is1 = (lg1 == m1[:, :, None]) & valid
    i1 = jnp.min(jnp.where(is1, lane, N_EXPERTS + 1), axis=2)
    oh1 = lane == i1[:, :, None]

    e1 = jnp.exp(m1 - m0)
    w0 = 1.0 / (1.0 + e1)
    w0_ref[...] = w0
    w1_ref[...] = 1.0 - w0

    p = jnp.exp(lg - m0[:, :, None])
    p = p / jnp.sum(p, axis=2, keepdims=True)
    usage = jnp.sum(p, axis=(0, 1)) * (1.0 / SEQ)
    loss = N_EXPERTS * jnp.sum(usage * usage) - 1.0
    loss_ref[...] = jnp.full((8, LANES), loss, jnp.float32)

    A0 = oh0.astype(jnp.float32)
    A1 = oh1.astype(jnp.float32)
    C = A0 + A1
    r_idx = lax.broadcasted_iota(jnp.int32, (CHUNKS, LANES, LANES), 1)
    p_idx = lax.broadcasted_iota(jnp.int32, (CHUNKS, LANES, LANES), 2)
    Tl = (p_idx < r_idx).astype(jnp.float32)
    X = lax.dot_general(Tl, C, (((2,), (1,)), ((0,), (0,))),
                        preferred_element_type=jnp.float32)
    tot = jnp.sum(C, axis=1)
    a16 = lax.broadcasted_iota(jnp.int32, (CHUNKS, CHUNKS), 0)
    b16 = lax.broadcasted_iota(jnp.int32, (CHUNKS, CHUNKS), 1)
    T16 = (b16 < a16).astype(jnp.float32)
    Y = lax.dot_general(T16, tot, (((1,), (0,)), ((), ())),
                        preferred_element_type=jnp.float32)
    S = X + Y[:, None, :]

    counts_row = jnp.sum(tot, axis=0)[None, :]
    pc_row = jnp.ceil(counts_row * (1.0 / TILE)) * TILE
    ag = lax.broadcasted_iota(jnp.int32, (LANES, LANES), 0)
    bg = lax.broadcasted_iota(jnp.int32, (LANES, LANES), 1)
    Tg = (ag < bg).astype(jnp.float32)
    starts_row = lax.dot_general(pc_row, Tg, (((1,), (0,)), ((), ())),
                                 preferred_element_type=jnp.float32)
    base = S + starts_row[0][None, None, :]
    pos0_ref[...] = jnp.sum(base * A0, axis=2).astype(jnp.int32)
    pos1_ref[...] = jnp.sum(base * A1, axis=2).astype(jnp.int32)

    Ti = (ag <= bg).astype(jnp.float32)
    ps_row = lax.dot_general(pc_row, Ti, (((1,), (0,)), ((), ())),
                             preferred_element_type=jnp.float32)
    total = jnp.sum(pc_row)
    used = total * (1.0 / TILE)
    lane_row = lax.broadcasted_iota(jnp.int32, (1, LANES), 1)
    estar = jnp.max(jnp.where((pc_row > 0) & (lane_row < N_EXPERTS),
                              lane_row.astype(jnp.float32), -1.0))
    nsub = lax.broadcasted_iota(jnp.int32, (LANES, LANES), 0).astype(
        jnp.float32)
    lane2 = lax.broadcasted_iota(jnp.int32, (LANES, LANES), 1)
    psb = jnp.broadcast_to(ps_row, (LANES, LANES))
    cmp = jnp.where((lane2 < N_EXPERTS) & (psb <= nsub * TILE), 1.0, 0.0)
    gidc = jnp.sum(cmp, axis=1, keepdims=True)
    nc = lax.broadcasted_iota(jnp.int32, (LANES, 1), 0).astype(jnp.float32)
    gidc = jnp.where(nc * TILE < total, gidc, estar)
    meta = jnp.where(nc == float(NT), used, gidc)
    meta_ref[...] = meta.astype(jnp.int32)


def _router(x2, gate_w, gate_b):
    gw = jnp.pad(gate_w, ((0, 0), (0, LANES - N_EXPERTS)))
    gb = jnp.pad(gate_b, (0, LANES - N_EXPERTS))[None, :]
    outs = pl.pallas_call(
        _router_body,
        out_shape=(
            jax.ShapeDtypeStruct((CHUNKS, LANES), jnp.int32),
            jax.ShapeDtypeStruct((CHUNKS, LANES), jnp.int32),
            jax.ShapeDtypeStruct((CHUNKS, LANES), jnp.float32),
            jax.ShapeDtypeStruct((CHUNKS, LANES), jnp.float32),
            jax.ShapeDtypeStruct((LANES, 1), jnp.int32),
            jax.ShapeDtypeStruct((8, LANES), jnp.float32),
        ),
    )(x2, gw, gb)
    return outs


def _dispatch_rows(x2, pos0, pos1):
    mesh = plsc.VectorSubcoreMesh(core_axis_name="c", subcore_axis_name="s")

    @functools.partial(
        pl.kernel, mesh=mesh,
        out_type=jax.ShapeDtypeStruct((PADDED, D_MODEL), jnp.float32),
        scratch_types=[
            pltpu.VMEM((ROWS_PER_W,), jnp.int32),
            pltpu.VMEM((ROWS_PER_W,), jnp.int32),
            pltpu.VMEM((ROWS_PER_W, D_MODEL), jnp.float32),
            pltpu.SemaphoreType.DMA,
            pltpu.SemaphoreType.DMA,
            pltpu.SemaphoreType.DMA,
        ],
    )
    def disp(x_hbm, pos0_hbm, pos1_hbm, xs_hbm, idx0_v, idx1_v, rows_v,
             s0, s1, s2):
        wid = lax.axis_index("s") * 2 + lax.axis_index("c")
        base = wid * ROWS_PER_W
        c0 = pltpu.async_copy(pos0_hbm.at[pl.ds(base, ROWS_PER_W)], idx0_v, s0)
        c1 = pltpu.async_copy(pos1_hbm.at[pl.ds(base, ROWS_PER_W)], idx1_v, s1)
        c2 = pltpu.async_copy(x_hbm.at[pl.ds(base, ROWS_PER_W), :], rows_v, s2)
        c0.wait()
        c1.wait()
        c2.wait()
        sc0 = pltpu.async_copy(rows_v, xs_hbm.at[idx0_v], s0)
        sc1 = pltpu.async_copy(rows_v, xs_hbm.at[idx1_v], s1)
        sc0.wait()
        sc1.wait()

    return disp(x2, pos0, pos1)


def _ffn_body(gid_ref, xs_ref, W1_ref, B1_ref, W2_ref, B2_ref, ys_ref):
    i = pl.program_id(0)

    @pl.when(i < gid_ref[NT])
    def _():
        h = jnp.dot(xs_ref[...], W1_ref[0], preferred_element_type=jnp.float32)
        h = h + B1_ref[0]
        h = 0.5 * h * (1.0 + lax.erf(h * 0.7071067811865476))
        y = jnp.dot(h, W2_ref[0], preferred_element_type=jnp.float32)
        ys_ref[...] = y + B2_ref[0]


def _ffn(xs, W1, B1, W2, B2, gid):
    grid_spec = pltpu.PrefetchScalarGridSpec(
        num_scalar_prefetch=1,
        grid=(NT,),
        in_specs=[
            pl.BlockSpec((TILE, D_MODEL), lambda i, g: (i, 0)),
            pl.BlockSpec((1, D_MODEL, D_FF), lambda i, g: (g[i], 0, 0)),
            pl.BlockSpec((1, 1, D_FF), lambda i, g: (g[i], 0, 0)),
            pl.BlockSpec((1, D_FF, D_MODEL), lambda i, g: (g[i], 0, 0)),
            pl.BlockSpec((1, 1, D_MODEL), lambda i, g: (g[i], 0, 0)),
        ],
        out_specs=pl.BlockSpec((TILE, D_MODEL), lambda i, g: (i, 0)),
    )
    return pl.pallas_call(
        _ffn_body,
        grid_spec=grid_spec,
        out_shape=jax.ShapeDtypeStruct((PADDED, D_MODEL), jnp.float32),
        compiler_params=pltpu.CompilerParams(
            dimension_semantics=("arbitrary",)),
    )(gid, xs, W1, B1[:, None, :], W2, B2[:, None, :])


def _combine_rows(ys, pos0, pos1, w0, w1):
    w0x = jnp.broadcast_to(w0[:, None], (SEQ, 16))
    w1x = jnp.broadcast_to(w1[:, None], (SEQ, 16))
    mesh = plsc.VectorSubcoreMesh(core_axis_name="c", subcore_axis_name="s")

    @functools.partial(
        pl.kernel, mesh=mesh,
        out_type=jax.ShapeDtypeStruct((SEQ, D_MODEL), jnp.float32),
        scratch_types=[
            pltpu.VMEM((ROWS_PER_W,), jnp.int32),
            pltpu.VMEM((ROWS_PER_W,), jnp.int32),
            pltpu.VMEM((ROWS_PER_W, 16), jnp.float32),
            pltpu.VMEM((ROWS_PER_W, 16), jnp.float32),
            pltpu.VMEM((ROWS_PER_W, D_MODEL), jnp.float32),
            pltpu.VMEM((ROWS_PER_W, D_MODEL), jnp.float32),
            pltpu.SemaphoreType.DMA,
            pltpu.SemaphoreType.DMA,
            pltpu.SemaphoreType.DMA,
            pltpu.SemaphoreType.DMA,
        ],
    )
    def comb(ys_hbm, pos0_hbm, pos1_hbm, w0_hbm, w1_hbm, out_hbm,
             i0_v, i1_v, w0_v, w1_v, r0_v, r1_v, s0, s1, s2, s3):
        wid = lax.axis_index("s") * 2 + lax.axis_index("c")
        base = wid * ROWS_PER_W
        c0 = pltpu.async_copy(pos0_hbm.at[pl.ds(base, ROWS_PER_W)], i0_v, s0)
        c1 = pltpu.async_copy(pos1_hbm.at[pl.ds(base, ROWS_PER_W)], i1_v, s1)
        cw0 = pltpu.async_copy(w0_hbm.at[pl.ds(base, ROWS_PER_W), :], w0_v, s2)
        cw1 = pltpu.async_copy(w1_hbm.at[pl.ds(base, ROWS_PER_W), :], w1_v, s3)
        c0.wait()
        g0 = pltpu.async_copy(ys_hbm.at[i0_v], r0_v, s0)
        c1.wait()
        g1 = pltpu.async_copy(ys_hbm.at[i1_v], r1_v, s1)
        cw0.wait()
        cw1.wait()
        g0.wait()
        g1.wait()

        def body(m, carry):
            a = w0_v[m, :]
            b = w1_v[m, :]
            for j in range(VREGS_PER_ROW):
                s = 16 * j
                r0_v[m, pl.ds(s, 16)] = (a * r0_v[m, pl.ds(s, 16)]
                                         + b * r1_v[m, pl.ds(s, 16)])
            return carry

        lax.fori_loop(0, ROWS_PER_W, body, 0)
        pltpu.sync_copy(r0_v, out_hbm.at[pl.ds(base, ROWS_PER_W), :])

    return comb(ys, pos0, pos1, w0x, w1x)


def kernel(x, gate_w, gate_b, W1, B1, W2, B2):
    x2 = x.reshape(SEQ, D_MODEL)
    pos0, pos1, w0, w1, meta, loss_m = _router(x2, gate_w, gate_b)
    pos0 = pos0.reshape(N_ASSIGN // 2)
    pos1 = pos1.reshape(N_ASSIGN // 2)
    w0 = w0.reshape(N_ASSIGN // 2)
    w1 = w1.reshape(N_ASSIGN // 2)
    gid = meta.reshape(LANES)

    xs = _dispatch_rows(x2, pos0, pos1)
    ys = _ffn(xs, W1, B1, W2, B2, gid)
    out = _combine_rows(ys, pos0, pos1, w0, w1)
    return out.reshape(1, SEQ, D_MODEL), loss_m[0, 0]


# speedup vs baseline: 1.2154x; 1.0063x over previous
import functools

import jax
import jax.numpy as jnp
from jax import lax
from jax.experimental import pallas as pl
from jax.experimental.pallas import tpu as pltpu
from jax.experimental.pallas import tpu_sc as plsc

D_MODEL = 768
D_FF = 3072
N_EXPERTS = 8
SEQ = 2048
LANES = 128
CHUNKS = SEQ // LANES
TILE = 256
N_ASSIGN = SEQ * 2
PADDED = N_ASSIGN + N_EXPERTS * TILE
NT = PADDED // TILE
NEG = -1e30

NW = 32
ROWS_PER_W = SEQ // NW
VREGS_PER_ROW = D_MODEL // 16


def _router_body(x_ref, gw_ref, gb_ref,
                 pos0_ref, pos1_ref, w0_ref, w1_ref, meta_ref, loss_ref):
    x = x_ref[...]
    logits2 = jnp.dot(x, gw_ref[...], preferred_element_type=jnp.float32)
    logits2 = logits2 + gb_ref[...]
    lg = logits2.reshape(CHUNKS, LANES, LANES)
    lane = lax.broadcasted_iota(jnp.int32, (CHUNKS, LANES, LANES), 2)
    valid = lane < N_EXPERTS
    lg = jnp.where(valid, lg, NEG)

    m0 = jnp.max(lg, axis=2)
    is0 = (lg == m0[:, :, None]) & valid
    i0 = jnp.min(jnp.where(is0, lane, N_EXPERTS + 1), axis=2)
    oh0 = lane == i0[:, :, None]
    lg1 = jnp.where(oh0, NEG, lg)
    m1 = jnp.max(lg1, axis=2)
    is1 = (lg1 == m1[:, :, None]) & valid
    i1 = jnp.min(jnp.where(is1, lane, N_EXPERTS + 1), axis=2)
    oh1 = lane == i1[:, :, None]

    e1 = jnp.exp(m1 - m0)
    w0 = 1.0 / (1.0 + e1)
    w0_ref[...] = w0
    w1_ref[...] = 1.0 - w0

    p = jnp.exp(lg - m0[:, :, None])
    p = p / jnp.sum(p, axis=2, keepdims=True)
    usage = jnp.sum(p, axis=(0, 1)) * (1.0 / SEQ)
    loss = N_EXPERTS * jnp.sum(usage * usage) - 1.0
    loss_ref[...] = jnp.full((8, LANES), loss, jnp.float32)

    A0 = oh0.astype(jnp.float32)
    A1 = oh1.astype(jnp.float32)
    C = A0 + A1
    r_idx = lax.broadcasted_iota(jnp.int32, (CHUNKS, LANES, LANES), 1)
    p_idx = lax.broadcasted_iota(jnp.int32, (CHUNKS, LANES, LANES), 2)
    Tl = (p_idx < r_idx).astype(jnp.float32)
    X = lax.dot_general(Tl, C, (((2,), (1,)), ((0,), (0,))),
                        preferred_element_type=jnp.float32)
    tot = jnp.sum(C, axis=1)
    a16 = lax.broadcasted_iota(jnp.int32, (CHUNKS, CHUNKS), 0)
    b16 = lax.broadcasted_iota(jnp.int32, (CHUNKS, CHUNKS), 1)
    T16 = (b16 < a16).astype(jnp.float32)
    Y = lax.dot_general(T16, tot, (((1,), (0,)), ((), ())),
                        preferred_element_type=jnp.float32)
    S = X + Y[:, None, :]

    counts_row = jnp.sum(tot, axis=0)[None, :]
    pc_row = jnp.ceil(counts_row * (1.0 / TILE)) * TILE
    ag = lax.broadcasted_iota(jnp.int32, (LANES, LANES), 0)
    bg = lax.broadcasted_iota(jnp.int32, (LANES, LANES), 1)
    Tg = (ag < bg).astype(jnp.float32)
    starts_row = lax.dot_general(pc_row, Tg, (((1,), (0,)), ((), ())),
                                 preferred_element_type=jnp.float32)
    base = S + starts_row[0][None, None, :]
    pos0_ref[...] = jnp.sum(base * A0, axis=2).astype(jnp.int32)
    pos1_ref[...] = jnp.sum(base * A1, axis=2).astype(jnp.int32)

    Ti = (ag <= bg).astype(jnp.float32)
    ps_row = lax.dot_general(pc_row, Ti, (((1,), (0,)), ((), ())),
                             preferred_element_type=jnp.float32)
    total = jnp.sum(pc_row)
    used = total * (1.0 / TILE)
    lane_row = lax.broadcasted_iota(jnp.int32, (1, LANES), 1)
    estar = jnp.max(jnp.where((pc_row > 0) & (lane_row < N_EXPERTS),
                              lane_row.astype(jnp.float32), -1.0))
    nsub = lax.broadcasted_iota(jnp.int32, (LANES, LANES), 0).astype(
        jnp.float32)
    lane2 = lax.broadcasted_iota(jnp.int32, (LANES, LANES), 1)
    psb = jnp.broadcast_to(ps_row, (LANES, LANES))
    cmp = jnp.where((lane2 < N_EXPERTS) & (psb <= nsub * TILE), 1.0, 0.0)
    gidc = jnp.sum(cmp, axis=1, keepdims=True)
    nc = lax.broadcasted_iota(jnp.int32, (LANES, 1), 0).astype(jnp.float32)
    gidc = jnp.where(nc * TILE < total, gidc, estar)
    meta = jnp.where(nc == float(NT), used, gidc)
    meta_ref[...] = meta.astype(jnp.int32)


def _router(x2, gate_w, gate_b):
    gw = jnp.pad(gate_w, ((0, 0), (0, LANES - N_EXPERTS)))
    gb = jnp.pad(gate_b, (0, LANES - N_EXPERTS))[None, :]
    outs = pl.pallas_call(
        _router_body,
        out_shape=(
            jax.ShapeDtypeStruct((CHUNKS, LANES), jnp.int32),
            jax.ShapeDtypeStruct((CHUNKS, LANES), jnp.int32),
            jax.ShapeDtypeStruct((CHUNKS, LANES), jnp.float32),
            jax.ShapeDtypeStruct((CHUNKS, LANES), jnp.float32),
            jax.ShapeDtypeStruct((LANES, 1), jnp.int32),
            jax.ShapeDtypeStruct((8, LANES), jnp.float32),
        ),
    )(x2, gw, gb)
    return outs


def _dispatch_rows(x2, pos0, pos1):
    mesh = plsc.VectorSubcoreMesh(core_axis_name="c", subcore_axis_name="s")

    @functools.partial(
        pl.kernel, mesh=mesh,
        out_type=jax.ShapeDtypeStruct((PADDED, D_MODEL), jnp.float32),
        scratch_types=[
            pltpu.VMEM((ROWS_PER_W,), jnp.int32),
            pltpu.VMEM((ROWS_PER_W,), jnp.int32),
            pltpu.VMEM((ROWS_PER_W, D_MODEL), jnp.float32),
            pltpu.SemaphoreType.DMA,
            pltpu.SemaphoreType.DMA,
            pltpu.SemaphoreType.DMA,
        ],
    )
    def disp(x_hbm, pos0_hbm, pos1_hbm, xs_hbm, idx0_v, idx1_v, rows_v,
             s0, s1, s2):
        wid = lax.axis_index("s") * 2 + lax.axis_index("c")
        base = wid * ROWS_PER_W
        c0 = pltpu.async_copy(pos0_hbm.at[pl.ds(base, ROWS_PER_W)], idx0_v, s0)
        c1 = pltpu.async_copy(pos1_hbm.at[pl.ds(base, ROWS_PER_W)], idx1_v, s1)
        c2 = pltpu.async_copy(x_hbm.at[pl.ds(base, ROWS_PER_W), :], rows_v, s2)
        c0.wait()
        c1.wait()
        c2.wait()
        sc0 = pltpu.async_copy(rows_v, xs_hbm.at[idx0_v], s0)
        sc1 = pltpu.async_copy(rows_v, xs_hbm.at[idx1_v], s1)
        sc0.wait()
        sc1.wait()

    return disp(x2, pos0, pos1)


def _ffn_body(gid_ref, xs_ref, W1_ref, B1_ref, W2_ref, B2_ref, ys_ref):
    i = pl.program_id(0)

    @pl.when(i < gid_ref[NT])
    def _():
        h = jnp.dot(xs_ref[...], W1_ref[0], preferred_element_type=jnp.float32)
        h = h + B1_ref[0]
        h = 0.5 * h * (1.0 + lax.erf(h * 0.7071067811865476))
        y = jnp.dot(h, W2_ref[0], preferred_element_type=jnp.float32)
        ys_ref[...] = y + B2_ref[0]


def _ffn(xs, W1, B1, W2, B2, gid):
    grid_spec = pltpu.PrefetchScalarGridSpec(
        num_scalar_prefetch=1,
        grid=(NT,),
        in_specs=[
            pl.BlockSpec((TILE, D_MODEL), lambda i, g: (i, 0)),
            pl.BlockSpec((1, D_MODEL, D_FF), lambda i, g: (g[i], 0, 0)),
            pl.BlockSpec((1, 1, D_FF), lambda i, g: (g[i], 0, 0)),
            pl.BlockSpec((1, D_FF, D_MODEL), lambda i, g: (g[i], 0, 0)),
            pl.BlockSpec((1, 1, D_MODEL), lambda i, g: (g[i], 0, 0)),
        ],
        out_specs=pl.BlockSpec((TILE, D_MODEL), lambda i, g: (i, 0)),
    )
    return pl.pallas_call(
        _ffn_body,
        grid_spec=grid_spec,
        out_shape=jax.ShapeDtypeStruct((PADDED, D_MODEL), jnp.float32),
        compiler_params=pltpu.CompilerParams(
            dimension_semantics=("arbitrary",)),
    )(gid, xs, W1, B1[:, None, :], W2, B2[:, None, :])


HALF = ROWS_PER_W // 2


def _combine_rows(ys, pos0, pos1, w0, w1):
    w0x = jnp.broadcast_to(w0[:, None], (SEQ, 16))
    w1x = jnp.broadcast_to(w1[:, None], (SEQ, 16))
    p0 = pos0.reshape(NW, 2, HALF)
    p1 = pos1.reshape(NW, 2, HALF)
    mesh = plsc.VectorSubcoreMesh(core_axis_name="c", subcore_axis_name="s")

    @functools.partial(
        pl.kernel, mesh=mesh,
        out_type=jax.ShapeDtypeStruct((SEQ, D_MODEL), jnp.float32),
        scratch_types=[
            pltpu.VMEM((2, HALF), jnp.int32),
            pltpu.VMEM((2, HALF), jnp.int32),
            pltpu.VMEM((ROWS_PER_W, 16), jnp.float32),
            pltpu.VMEM((ROWS_PER_W, 16), jnp.float32),
            pltpu.VMEM((ROWS_PER_W, D_MODEL), jnp.float32),
            pltpu.VMEM((ROWS_PER_W, D_MODEL), jnp.float32),
            pltpu.SemaphoreType.DMA,
            pltpu.SemaphoreType.DMA,
            pltpu.SemaphoreType.DMA,
            pltpu.SemaphoreType.DMA,
            pltpu.SemaphoreType.DMA,
            pltpu.SemaphoreType.DMA,
            pltpu.SemaphoreType.DMA,
        ],
    )
    def comb(ys_hbm, pos0_hbm, pos1_hbm, w0_hbm, w1_hbm, out_hbm,
             i0_v, i1_v, w0_v, w1_v, r0_v, r1_v, s0, s1, s2, s3, s4, s5, s6):
        wid = lax.axis_index("s") * 2 + lax.axis_index("c")
        base = wid * ROWS_PER_W
        c0 = pltpu.async_copy(pos0_hbm.at[wid], i0_v, s0)
        c1 = pltpu.async_copy(pos1_hbm.at[wid], i1_v, s1)
        cw0 = pltpu.async_copy(w0_hbm.at[pl.ds(base, ROWS_PER_W), :], w0_v, s2)
        cw1 = pltpu.async_copy(w1_hbm.at[pl.ds(base, ROWS_PER_W), :], w1_v, s3)
        c0.wait()
        c1.wait()
        g00 = pltpu.async_copy(ys_hbm.at[i0_v.at[0]],
                               r0_v.at[pl.ds(0, HALF), :], s0)
        g10 = pltpu.async_copy(ys_hbm.at[i1_v.at[0]],
                               r1_v.at[pl.ds(0, HALF), :], s1)
        g01 = pltpu.async_copy(ys_hbm.at[i0_v.at[1]],
                               r0_v.at[pl.ds(HALF, HALF), :], s4)
        g11 = pltpu.async_copy(ys_hbm.at[i1_v.at[1]],
                               r1_v.at[pl.ds(HALF, HALF), :], s5)
        cw0.wait()
        cw1.wait()

        def body(m, carry):
            a = w0_v[m, :]
            b = w1_v[m, :]
            for j in range(VREGS_PER_ROW):
                s = 16 * j
                r0_v[m, pl.ds(s, 16)] = (a * r0_v[m, pl.ds(s, 16)]
                                         + b * r1_v[m, pl.ds(s, 16)])
            return carry

        g00.wait()
        g10.wait()
        lax.fori_loop(0, HALF, body, 0)
        wb0 = pltpu.async_copy(r0_v.at[pl.ds(0, HALF), :],
                               out_hbm.at[pl.ds(base, HALF), :], s6)
        g01.wait()
        g11.wait()
        lax.fori_loop(HALF, ROWS_PER_W, body, 0)
        wb1 = pltpu.async_copy(r0_v.at[pl.ds(HALF, HALF), :],
                               out_hbm.at[pl.ds(base + HALF, HALF), :], s2)
        wb0.wait()
        wb1.wait()

    return comb(ys, p0, p1, w0x, w1x)


def kernel(x, gate_w, gate_b, W1, B1, W2, B2):
    x2 = x.reshape(SEQ, D_MODEL)
    pos0, pos1, w0, w1, meta, loss_m = _router(x2, gate_w, gate_b)
    pos0 = pos0.reshape(N_ASSIGN // 2)
    pos1 = pos1.reshape(N_ASSIGN // 2)
    w0 = w0.reshape(N_ASSIGN // 2)
    w1 = w1.reshape(N_ASSIGN // 2)
    gid = meta.reshape(LANES)

    xs = _dispatch_rows(x2, pos0, pos1)
    ys = _ffn(xs, W1, B1, W2, B2, gid)
    out = _combine_rows(ys, pos0, pos1, w0, w1)
    return out.reshape(1, SEQ, D_MODEL), loss_m[0, 0]
